# Initial kernel scaffold; baseline (speedup 1.0000x reference)
import jax
import jax.numpy as jnp
from jax.experimental import pallas as pl


def kernel(x, w_dkv, w_uk, w_uv, w_dq, w_uq, w_qr, w_kr, raw_delta, w_out, idx_wq, idx_wk, idx_w):
    raise NotImplementedError("write your pallas kernel here")


# speedup vs baseline: 1.3724x; 1.3724x over previous
import functools

import jax
import jax.numpy as jnp
from jax import lax
from jax.experimental import pallas as pl
from jax.experimental.pallas import tpu as pltpu
from jax.experimental.pallas import tpu_sc as plsc

L = 2048
D_MODEL = 768
D_CKV = 192
D_CQ = 256
N_HEAD = 8
D_HEAD = 64
D_ROPE = 32
HALF = D_ROPE // 2
K_TS = 32
LOCAL_WINDOW = 16
N_IDX_HEADS = 2
ROPE_BASE = 10000.0
D_TAB = D_CKV + D_ROPE
D_PAD = 256

BLK = 256
GRID = L // BLK
BLK_A = 64
GRID_A = L // BLK_A

_ENC_PINF = 0x7F800000
_ENC_NINF = -2139095041
_SENTINEL = -2147483648


def _softplus(v):
    return jnp.maximum(v, 0.0) + jnp.log(1.0 + jnp.exp(-jnp.abs(v)))


def _prep_body(x_ref, w_dkv_ref, w_kr_ref, w_dq_ref, w_uq_ref, w_qr_ref,
               w_uk_ref, idx_wq_ref, idx_wk_ref, cos_q_ref, sin_q_ref,
               table_ref, qlat_ref, qrot_ref, qi_ref, ki_ref):
    x = x_ref[...]
    c_kv = jnp.dot(x, w_dkv_ref[...], preferred_element_type=jnp.float32, precision=jax.lax.Precision.HIGHEST)
    kr_sp = _softplus(jnp.dot(x, w_kr_ref[...],
                              preferred_element_type=jnp.float32, precision=jax.lax.Precision.HIGHEST))
    table_ref[...] = jnp.concatenate(
        [c_kv, kr_sp, jnp.zeros((BLK, D_PAD - D_TAB), jnp.float32)], axis=1)

    c_q = jnp.dot(x, w_dq_ref[...], preferred_element_type=jnp.float32, precision=jax.lax.Precision.HIGHEST)
    q_c = jnp.dot(c_q, w_uq_ref[...], preferred_element_type=jnp.float32, precision=jax.lax.Precision.HIGHEST)
    q_r = jnp.dot(c_q, w_qr_ref[...], preferred_element_type=jnp.float32, precision=jax.lax.Precision.HIGHEST)
    w_uk = w_uk_ref[...]
    cos_q = cos_q_ref[...]
    sin_q = sin_q_ref[...]

    qlat_parts = []
    qrot_parts = []
    for h in range(N_HEAD):
        qc_h = q_c[:, h * D_HEAD:(h + 1) * D_HEAD]
        wuk_h = w_uk[:, h * D_HEAD:(h + 1) * D_HEAD]
        qlat_parts.append(lax.dot_general(
            qc_h, wuk_h, (((1,), (1,)), ((), ())),
            preferred_element_type=jnp.float32, precision=jax.lax.Precision.HIGHEST))
        sp = _softplus(q_r[:, h * D_ROPE:(h + 1) * D_ROPE])
        mu1 = sp[:, :HALF]
        mu2 = sp[:, HALF:]
        qrot_parts.append(mu1 * cos_q - mu2 * sin_q)
        qrot_parts.append(mu1 * sin_q + mu2 * cos_q)
    qlat_ref[...] = jnp.concatenate(qlat_parts, axis=1)
    qrot_ref[...] = jnp.concatenate(qrot_parts, axis=1)

    qi_ref[...] = jnp.dot(x, idx_wq_ref[...], preferred_element_type=jnp.float32)
    ki_ref[...] = jnp.dot(x, idx_wk_ref[...], preferred_element_type=jnp.float32)


def _topk_body(qi_ref, ki_ref, iw_ref, idx_ref):
    blk = pl.program_id(0)
    q_i = qi_ref[...]
    k_i = ki_ref[...]
    iw = iw_ref[...]

    I = jnp.zeros((BLK, L), jnp.float32)
    for h in range(N_IDX_HEADS):
        s = lax.dot_general(q_i[:, h * D_HEAD:(h + 1) * D_HEAD], k_i,
                            (((1,), (1,)), ((), ())),
                            preferred_element_type=jnp.float32)
        r = jnp.maximum(s, 0.0).astype(jnp.bfloat16).astype(jnp.float32)
        w = iw[:, h:h + 1].astype(jnp.bfloat16).astype(jnp.float32)
        I = I + r * w

    row = blk * BLK + lax.broadcasted_iota(jnp.int32, (BLK, L), 0)
    col = lax.broadcasted_iota(jnp.int32, (BLK, L), 1)
    local = (col >= row - (LOCAL_WINDOW - 1)) & (col <= row)
    causal = col > row

    b = lax.bitcast_convert_type(I, jnp.int32)
    enc = jnp.where(b < 0, b ^ jnp.int32(0x7FFFFFFF), b)
    enc = jnp.where(causal, jnp.int32(_ENC_NINF), enc)
    enc = jnp.where(local, jnp.int32(_ENC_PINF), enc)

    cols_out = []
    for _ in range(K_TS):
        m = jnp.max(enc, axis=1, keepdims=True)
        cand = jnp.where(enc == m, col, jnp.int32(L))
        j = jnp.min(cand, axis=1, keepdims=True)
        cols_out.append(j)
        enc = jnp.where(col == j, jnp.int32(_SENTINEL), enc)
    idx_ref[...] = jnp.concatenate(cols_out, axis=1)


def _attn_body(g_ref, qlat_ref, qrot_ref, cos_k_ref, sin_k_ref,
               w_uv_ref, w_out_ref, out_ref):
    g = g_ref[...]
    ckv = g[:, :, :D_CKV]
    mu1 = g[:, :, D_CKV:D_CKV + HALF]
    mu2 = g[:, :, D_CKV + HALF:D_TAB]
    cos_k = cos_k_ref[...][None, :, :]
    sin_k = sin_k_ref[...][None, :, :]
    kr1 = mu1 * cos_k - mu2 * sin_k
    kr2 = mu1 * sin_k + mu2 * cos_k

    qlat = qlat_ref[...]
    qrot = qrot_ref[...]
    w_uv = w_uv_ref[...]
    scale = (D_HEAD + D_ROPE) ** -0.5

    o_parts = []
    for h in range(N_HEAD):
        ql_h = qlat[:, h * D_CKV:(h + 1) * D_CKV]
        qr1 = qrot[:, h * D_ROPE:h * D_ROPE + HALF]
        qr2 = qrot[:, h * D_ROPE + HALF:(h + 1) * D_ROPE]
        s = jnp.sum(ql_h[:, None, :] * ckv, axis=2)
        s = s + jnp.sum(qr1[:, None, :] * kr1, axis=2)
        s = s + jnp.sum(qr2[:, None, :] * kr2, axis=2)
        s = s * scale
        m = jnp.max(s, axis=1, keepdims=True)
        e = jnp.exp(s - m)
        p = e / jnp.sum(e, axis=1, keepdims=True)
        w_lat = jnp.sum(p[:, :, None] * ckv, axis=1)
        o_parts.append(jnp.dot(w_lat, w_uv[:, h * D_HEAD:(h + 1) * D_HEAD],
                               preferred_element_type=jnp.float32, precision=jax.lax.Precision.HIGHEST))
    attn_o = jnp.concatenate(o_parts, axis=1)
    out_ref[...] = jnp.dot(attn_o, w_out_ref[...],
                           preferred_element_type=jnp.float32, precision=jax.lax.Precision.HIGHEST)


_NW = 32
_BPW = (L * K_TS) // _NW
_CH = 128


def _sc_gather_call(table, idx_flat):
    mesh = plsc.VectorSubcoreMesh(core_axis_name="c", subcore_axis_name="s")

    @functools.partial(
        pl.kernel, mesh=mesh,
        out_type=jax.ShapeDtypeStruct((L * K_TS, D_PAD), jnp.float32),
        scratch_types=[
            pltpu.VMEM((_CH,), jnp.int32),
            pltpu.VMEM((_CH, D_PAD), jnp.float32),
            pltpu.SemaphoreType.DMA,
        ],
    )
    def gk(table_hbm, idx_hbm, out_hbm, idx_v, rows_v, sem):
        wid = lax.axis_index("s") * 2 + lax.axis_index("c")
        base = wid * _BPW

        def body(i, carry):
            off = base + i * _CH
            pltpu.sync_copy(idx_hbm.at[pl.ds(off, _CH)], idx_v)
            pltpu.async_copy(table_hbm.at[idx_v], rows_v, sem).wait()
            pltpu.sync_copy(rows_v, out_hbm.at[pl.ds(off, _CH)])
            return carry

        lax.fori_loop(0, _BPW // _CH, body, 0)

    return gk(table, idx_flat)


def _prep_call(x2, w_dkv, w_kr, w_dq, w_uq, w_qr, w_uk, idx_wq, idx_wk,
               cos_q, sin_q):
    full = lambda r, c: pl.BlockSpec((r, c), lambda i: (0, 0))
    blkd = lambda c: pl.BlockSpec((BLK, c), lambda i: (i, 0))
    return pl.pallas_call(
        _prep_body,
        grid=(GRID,),
        in_specs=[
            blkd(D_MODEL),
            full(D_MODEL, D_CKV),
            full(D_MODEL, D_ROPE),
            full(D_MODEL, D_CQ),
            full(D_CQ, N_HEAD * D_HEAD),
            full(D_CQ, N_HEAD * D_ROPE),
            full(D_CKV, N_HEAD * D_HEAD),
            full(D_MODEL, N_IDX_HEADS * D_HEAD),
            full(D_MODEL, D_HEAD),
            blkd(HALF),
            blkd(HALF),
        ],
        out_specs=[
            blkd(D_PAD),
            blkd(N_HEAD * D_CKV),
            blkd(N_HEAD * D_ROPE),
            blkd(N_IDX_HEADS * D_HEAD),
            blkd(D_HEAD),
        ],
        out_shape=[
            jax.ShapeDtypeStruct((L, D_PAD), jnp.float32),
            jax.ShapeDtypeStruct((L, N_HEAD * D_CKV), jnp.float32),
            jax.ShapeDtypeStruct((L, N_HEAD * D_ROPE), jnp.float32),
            jax.ShapeDtypeStruct((L, N_IDX_HEADS * D_HEAD), jnp.float32),
            jax.ShapeDtypeStruct((L, D_HEAD), jnp.float32),
        ],
    )(x2, w_dkv, w_kr, w_dq, w_uq, w_qr, w_uk, idx_wq, idx_wk, cos_q, sin_q)


def _topk_call(q_i, k_i, iw):
    return pl.pallas_call(
        _topk_body,
        grid=(GRID,),
        in_specs=[
            pl.BlockSpec((BLK, N_IDX_HEADS * D_HEAD), lambda i: (i, 0)),
            pl.BlockSpec((L, D_HEAD), lambda i: (0, 0)),
            pl.BlockSpec((1, N_IDX_HEADS), lambda i: (0, 0)),
        ],
        out_specs=pl.BlockSpec((BLK, K_TS), lambda i: (i, 0)),
        out_shape=jax.ShapeDtypeStruct((L, K_TS), jnp.int32),
    )(q_i, k_i, iw)


def _attn_call(g3, qlat, qrot, cos_k, sin_k, w_uv, w_out):
    return pl.pallas_call(
        _attn_body,
        grid=(GRID_A,),
        in_specs=[
            pl.BlockSpec((BLK_A, K_TS, D_PAD), lambda i: (i, 0, 0)),
            pl.BlockSpec((BLK_A, N_HEAD * D_CKV), lambda i: (i, 0)),
            pl.BlockSpec((BLK_A, N_HEAD * D_ROPE), lambda i: (i, 0)),
            pl.BlockSpec((K_TS, HALF), lambda i: (0, 0)),
            pl.BlockSpec((K_TS, HALF), lambda i: (0, 0)),
            pl.BlockSpec((D_CKV, N_HEAD * D_HEAD), lambda i: (0, 0)),
            pl.BlockSpec((N_HEAD * D_HEAD, D_MODEL), lambda i: (0, 0)),
        ],
        out_specs=pl.BlockSpec((BLK_A, D_MODEL), lambda i: (i, 0)),
        out_shape=jax.ShapeDtypeStruct((L, D_MODEL), jnp.float32),
    )(g3, qlat, qrot, cos_k, sin_k, w_uv, w_out)


def kernel(x, w_dkv, w_uk, w_uv, w_dq, w_uq, w_qr, w_kr, raw_delta, w_out,
           idx_wq, idx_wk, idx_w):
    b = x.shape[0]
    x2 = x.reshape(L, D_MODEL)

    theta = 1.0 / (ROPE_BASE ** (2.0 * jnp.arange(HALF, dtype=jnp.float32)
                                 / D_ROPE))
    delta = -2.0 * jnp.pi * jax.nn.sigmoid(raw_delta)
    ang_q = jnp.arange(L, dtype=jnp.float32)[:, None] * theta[None, :] + delta[None, :]
    ang_k = jnp.arange(K_TS, dtype=jnp.float32)[:, None] * theta[None, :] + delta[None, :]
    cos_q, sin_q = jnp.cos(ang_q), jnp.sin(ang_q)
    cos_k, sin_k = jnp.cos(ang_k), jnp.sin(ang_k)

    table, qlat, qrot, q_i, k_i = _prep_call(
        x2, w_dkv, w_kr, w_dq, w_uq, w_qr, w_uk, idx_wq, idx_wk, cos_q, sin_q)
    idx = _topk_call(q_i, k_i, idx_w.reshape(1, N_IDX_HEADS))
    gathered = _sc_gather_call(table, idx.reshape(-1))
    out = _attn_call(gathered.reshape(L, K_TS, D_PAD), qlat, qrot,
                     cos_k, sin_k, w_uv, w_out)
    return out.reshape(b, L, D_MODEL)


# speedup vs baseline: 1.6132x; 1.1755x over previous
import functools

import jax
import jax.numpy as jnp
from jax import lax
from jax.experimental import pallas as pl
from jax.experimental.pallas import tpu as pltpu
from jax.experimental.pallas import tpu_sc as plsc

L = 2048
D_MODEL = 768
D_CKV = 192
D_CQ = 256
N_HEAD = 8
D_HEAD = 64
D_ROPE = 32
HALF = D_ROPE // 2
K_TS = 32
LOCAL_WINDOW = 16
N_IDX_HEADS = 2
ROPE_BASE = 10000.0
D_TAB = D_CKV + D_ROPE
D_PAD = 256

BLK = 256
GRID = L // BLK
BLK_A = 64
GRID_A = L // BLK_A

_ENC_PINF = 0x7F800000
_ENC_NINF = -2139095041
_SENTINEL = -2147483648


def _softplus(v):
    return jnp.maximum(v, 0.0) + jnp.log(1.0 + jnp.exp(-jnp.abs(v)))


def _prep_body(x_ref, w_dkv_ref, w_kr_ref, w_dq_ref, w_uq_ref, w_qr_ref,
               w_uk_ref, idx_wq_ref, idx_wk_ref, cos_q_ref, sin_q_ref,
               table_ref, qlat_ref, qrot_ref, qi_ref, ki_ref):
    x = x_ref[...]
    c_kv = jnp.dot(x, w_dkv_ref[...], preferred_element_type=jnp.float32)
    kr_sp = _softplus(jnp.dot(x, w_kr_ref[...],
                              preferred_element_type=jnp.float32))
    table_ref[...] = jnp.concatenate(
        [c_kv, kr_sp, jnp.zeros((BLK, D_PAD - D_TAB), jnp.float32)], axis=1)

    c_q = jnp.dot(x, w_dq_ref[...], preferred_element_type=jnp.float32)
    q_c = jnp.dot(c_q, w_uq_ref[...], preferred_element_type=jnp.float32)
    q_r = jnp.dot(c_q, w_qr_ref[...], preferred_element_type=jnp.float32)
    w_uk = w_uk_ref[...]
    cos_q = cos_q_ref[...]
    sin_q = sin_q_ref[...]

    qlat_parts = []
    qrot_parts = []
    for h in range(N_HEAD):
        qc_h = q_c[:, h * D_HEAD:(h + 1) * D_HEAD]
        wuk_h = w_uk[:, h * D_HEAD:(h + 1) * D_HEAD]
        qlat_parts.append(lax.dot_general(
            qc_h, wuk_h, (((1,), (1,)), ((), ())),
            preferred_element_type=jnp.float32))
        sp = _softplus(q_r[:, h * D_ROPE:(h + 1) * D_ROPE])
        mu1 = sp[:, :HALF]
        mu2 = sp[:, HALF:]
        qrot_parts.append(mu1 * cos_q - mu2 * sin_q)
        qrot_parts.append(mu1 * sin_q + mu2 * cos_q)
    qlat_ref[...] = jnp.concatenate(qlat_parts, axis=1)
    qrot_ref[...] = jnp.concatenate(qrot_parts, axis=1)

    qi_ref[...] = jnp.dot(x, idx_wq_ref[...], preferred_element_type=jnp.float32)
    ki_ref[...] = jnp.dot(x, idx_wk_ref[...], preferred_element_type=jnp.float32)


def _topk_body(qi_ref, ki_ref, iw_ref, idx_ref):
    blk = pl.program_id(0)
    q_i = qi_ref[...]
    k_i = ki_ref[...]
    iw = iw_ref[...]

    I = jnp.zeros((BLK, L), jnp.float32)
    for h in range(N_IDX_HEADS):
        s = lax.dot_general(q_i[:, h * D_HEAD:(h + 1) * D_HEAD], k_i,
                            (((1,), (1,)), ((), ())),
                            preferred_element_type=jnp.float32)
        r = jnp.maximum(s, 0.0).astype(jnp.bfloat16).astype(jnp.float32)
        w = iw[:, h:h + 1].astype(jnp.bfloat16).astype(jnp.float32)
        I = I + r * w

    row = blk * BLK + lax.broadcasted_iota(jnp.int32, (BLK, L), 0)
    col = lax.broadcasted_iota(jnp.int32, (BLK, L), 1)

    row16 = blk * BLK + lax.broadcasted_iota(jnp.int32, (BLK, LOCAL_WINDOW), 0)
    k16 = lax.broadcasted_iota(jnp.int32, (BLK, LOCAL_WINDOW), 1)
    first16 = k16 + jnp.maximum(row16 - (LOCAL_WINDOW - 1), 0)

    b = lax.bitcast_convert_type(I, jnp.int32)
    enc = jnp.where(b < 0, b ^ jnp.int32(0x7FFFFFFF), b)
    enc = jnp.where(col > row, jnp.int32(_ENC_NINF), enc)
    picked = (col >= row - (LOCAL_WINDOW - 1)) & (col <= jnp.maximum(row, LOCAL_WINDOW - 1))
    enc = jnp.where(picked, jnp.int32(_SENTINEL), enc)

    cols_out = [first16]
    for _ in range(K_TS - LOCAL_WINDOW):
        m = jnp.max(enc, axis=1, keepdims=True)
        cand = jnp.where(enc == m, col, jnp.int32(L))
        j = jnp.min(cand, axis=1, keepdims=True)
        cols_out.append(j)
        enc = jnp.where(col == j, jnp.int32(_SENTINEL), enc)
    idx_ref[...] = jnp.concatenate(cols_out, axis=1)


def _attn_body(g_ref, qlat_ref, qrot_ref, cos_k_ref, sin_k_ref,
               w_uv_ref, w_out_ref, out_ref):
    g = g_ref[...]
    ckv = g[:, :, :D_CKV]
    mu1 = g[:, :, D_CKV:D_CKV + HALF]
    mu2 = g[:, :, D_CKV + HALF:D_TAB]
    cos_k = cos_k_ref[...][None, :, :]
    sin_k = sin_k_ref[...][None, :, :]
    kr1 = mu1 * cos_k - mu2 * sin_k
    kr2 = mu1 * sin_k + mu2 * cos_k

    qlat = qlat_ref[...]
    qrot = qrot_ref[...]
    w_uv = w_uv_ref[...]
    scale = (D_HEAD + D_ROPE) ** -0.5

    o_parts = []
    for h in range(N_HEAD):
        ql_h = qlat[:, h * D_CKV:(h + 1) * D_CKV]
        qr1 = qrot[:, h * D_ROPE:h * D_ROPE + HALF]
        qr2 = qrot[:, h * D_ROPE + HALF:(h + 1) * D_ROPE]
        s = jnp.sum(ql_h[:, None, :] * ckv, axis=2)
        s = s + jnp.sum(qr1[:, None, :] * kr1, axis=2)
        s = s + jnp.sum(qr2[:, None, :] * kr2, axis=2)
        s = s * scale
        m = jnp.max(s, axis=1, keepdims=True)
        e = jnp.exp(s - m)
        p = e / jnp.sum(e, axis=1, keepdims=True)
        w_lat = jnp.sum(p[:, :, None] * ckv, axis=1)
        o_parts.append(jnp.dot(w_lat, w_uv[:, h * D_HEAD:(h + 1) * D_HEAD],
                               preferred_element_type=jnp.float32))
    attn_o = jnp.concatenate(o_parts, axis=1)
    out_ref[...] = jnp.dot(attn_o, w_out_ref[...],
                           preferred_element_type=jnp.float32)


_NW = 32
_BPW = (L * K_TS) // _NW
_CH = 128


def _sc_gather_call(table, idx_flat):
    mesh = plsc.VectorSubcoreMesh(core_axis_name="c", subcore_axis_name="s")

    @functools.partial(
        pl.kernel, mesh=mesh,
        out_type=jax.ShapeDtypeStruct((L * K_TS, D_PAD), jnp.float32),
        scratch_types=[
            pltpu.VMEM((_CH,), jnp.int32),
            pltpu.VMEM((_CH, D_PAD), jnp.float32),
            pltpu.SemaphoreType.DMA,
        ],
    )
    def gk(table_hbm, idx_hbm, out_hbm, idx_v, rows_v, sem):
        wid = lax.axis_index("s") * 2 + lax.axis_index("c")
        base = wid * _BPW

        def body(i, carry):
            off = base + i * _CH
            pltpu.sync_copy(idx_hbm.at[pl.ds(off, _CH)], idx_v)
            pltpu.async_copy(table_hbm.at[idx_v], rows_v, sem).wait()
            pltpu.sync_copy(rows_v, out_hbm.at[pl.ds(off, _CH)])
            return carry

        lax.fori_loop(0, _BPW // _CH, body, 0)

    return gk(table, idx_flat)


def _prep_call(x2, w_dkv, w_kr, w_dq, w_uq, w_qr, w_uk, idx_wq, idx_wk,
               cos_q, sin_q):
    full = lambda r, c: pl.BlockSpec((r, c), lambda i: (0, 0))
    blkd = lambda c: pl.BlockSpec((BLK, c), lambda i: (i, 0))
    return pl.pallas_call(
        _prep_body,
        grid=(GRID,),
        in_specs=[
            blkd(D_MODEL),
            full(D_MODEL, D_CKV),
            full(D_MODEL, D_ROPE),
            full(D_MODEL, D_CQ),
            full(D_CQ, N_HEAD * D_HEAD),
            full(D_CQ, N_HEAD * D_ROPE),
            full(D_CKV, N_HEAD * D_HEAD),
            full(D_MODEL, N_IDX_HEADS * D_HEAD),
            full(D_MODEL, D_HEAD),
            blkd(HALF),
            blkd(HALF),
        ],
        out_specs=[
            blkd(D_PAD),
            blkd(N_HEAD * D_CKV),
            blkd(N_HEAD * D_ROPE),
            blkd(N_IDX_HEADS * D_HEAD),
            blkd(D_HEAD),
        ],
        out_shape=[
            jax.ShapeDtypeStruct((L, D_PAD), jnp.float32),
            jax.ShapeDtypeStruct((L, N_HEAD * D_CKV), jnp.float32),
            jax.ShapeDtypeStruct((L, N_HEAD * D_ROPE), jnp.float32),
            jax.ShapeDtypeStruct((L, N_IDX_HEADS * D_HEAD), jnp.float32),
            jax.ShapeDtypeStruct((L, D_HEAD), jnp.float32),
        ],
    )(x2, w_dkv, w_kr, w_dq, w_uq, w_qr, w_uk, idx_wq, idx_wk, cos_q, sin_q)


def _topk_call(q_i, k_i, iw):
    return pl.pallas_call(
        _topk_body,
        grid=(GRID,),
        in_specs=[
            pl.BlockSpec((BLK, N_IDX_HEADS * D_HEAD), lambda i: (i, 0)),
            pl.BlockSpec((L, D_HEAD), lambda i: (0, 0)),
            pl.BlockSpec((1, N_IDX_HEADS), lambda i: (0, 0)),
        ],
        out_specs=pl.BlockSpec((BLK, K_TS), lambda i: (i, 0)),
        out_shape=jax.ShapeDtypeStruct((L, K_TS), jnp.int32),
    )(q_i, k_i, iw)


def _attn_call(g3, qlat, qrot, cos_k, sin_k, w_uv, w_out):
    return pl.pallas_call(
        _attn_body,
        grid=(GRID_A,),
        in_specs=[
            pl.BlockSpec((BLK_A, K_TS, D_PAD), lambda i: (i, 0, 0)),
            pl.BlockSpec((BLK_A, N_HEAD * D_CKV), lambda i: (i, 0)),
            pl.BlockSpec((BLK_A, N_HEAD * D_ROPE), lambda i: (i, 0)),
            pl.BlockSpec((K_TS, HALF), lambda i: (0, 0)),
            pl.BlockSpec((K_TS, HALF), lambda i: (0, 0)),
            pl.BlockSpec((D_CKV, N_HEAD * D_HEAD), lambda i: (0, 0)),
            pl.BlockSpec((N_HEAD * D_HEAD, D_MODEL), lambda i: (0, 0)),
        ],
        out_specs=pl.BlockSpec((BLK_A, D_MODEL), lambda i: (i, 0)),
        out_shape=jax.ShapeDtypeStruct((L, D_MODEL), jnp.float32),
    )(g3, qlat, qrot, cos_k, sin_k, w_uv, w_out)


def kernel(x, w_dkv, w_uk, w_uv, w_dq, w_uq, w_qr, w_kr, raw_delta, w_out,
           idx_wq, idx_wk, idx_w):
    b = x.shape[0]
    x2 = x.reshape(L, D_MODEL)

    theta = 1.0 / (ROPE_BASE ** (2.0 * jnp.arange(HALF, dtype=jnp.float32)
                                 / D_ROPE))
    delta = -2.0 * jnp.pi * jax.nn.sigmoid(raw_delta)
    ang_q = jnp.arange(L, dtype=jnp.float32)[:, None] * theta[None, :] + delta[None, :]
    ang_k = jnp.arange(K_TS, dtype=jnp.float32)[:, None] * theta[None, :] + delta[None, :]
    cos_q, sin_q = jnp.cos(ang_q), jnp.sin(ang_q)
    cos_k, sin_k = jnp.cos(ang_k), jnp.sin(ang_k)

    table, qlat, qrot, q_i, k_i = _prep_call(
        x2, w_dkv, w_kr, w_dq, w_uq, w_qr, w_uk, idx_wq, idx_wk, cos_q, sin_q)
    idx = _topk_call(q_i, k_i, idx_w.reshape(1, N_IDX_HEADS))
    gathered = _sc_gather_call(table, idx.reshape(-1))
    out = _attn_call(gathered.reshape(L, K_TS, D_PAD), qlat, qrot,
                     cos_k, sin_k, w_uv, w_out)
    return out.reshape(b, L, D_MODEL)
